# baseline (speedup 1.0000x reference)
import jax
import jax.numpy as jnp
from jax import lax
from jax.experimental import pallas as pl
from jax.experimental.pallas import tpu as pltpu
from jax.experimental.pallas import tpu_sc as plsc

N_NODES = 10000
N_PAD = 10240
D = 128
HD = D // 2
E = 320000
CHUNK = 128
NC = 2
NS = 16
ROWS_PER_TILE = N_PAD // NS
WB = 80

E_PAD = 327680
N_CHUNKS = E_PAD // CHUNK
TC_CHUNKS = N_CHUNKS // NS
SLAB = 8
N_SLABS = TC_CHUNKS // SLAB


def _rsqrt16(d):
    ii = lax.bitcast_convert_type(d, jnp.int32)
    ii = jnp.int32(0x5F3759DF) - lax.shift_right_logical(ii, 1)
    y = lax.bitcast_convert_type(ii, jnp.float32)
    for _ in range(3):
        y = y * (1.5 - 0.5 * d * y * y)
    return jnp.where(d > 0.0, y, 0.0)


def _sc_body(x2_hbm, row2d, col2d, ew2d, part_hbm,
             rowb, ewb, colslab, normbuf, rows0, rows1, dinv_l, ones,
             zerobuf, zbuf1, deg_sh, acc_sh,
             sem_g0, sem_g1, sem_s0, sem_s1, sem_c0, sem_c1, sem_z):
    c = lax.axis_index("c")
    t = lax.axis_index("s")
    r0 = t * ROWS_PER_TILE
    cbase = t * TC_CHUNKS
    sbase = t * N_SLABS

    zvec = jnp.zeros((16,), jnp.float32)
    ovec = jnp.ones((16,), jnp.float32)
    for j in range(CHUNK // 16):
        ones[pl.ds(j * 16, 16)] = ovec

    def _zrow(i, _):
        for j in range(HD // 16):
            zerobuf[i, pl.ds(j * 16, 16)] = zvec
        return _
    lax.fori_loop(0, WB, _zrow, None)

    def _z1(i, _):
        zbuf1[pl.ds(i * 16, 16)] = zvec
        return _
    lax.fori_loop(0, ROWS_PER_TILE // 16, _z1, None)

    pltpu.sync_copy(row2d.at[pl.ds(cbase, TC_CHUNKS)], rowb)
    pltpu.sync_copy(ew2d.at[pl.ds(cbase, TC_CHUNKS)], ewb)

    for k in range(ROWS_PER_TILE // WB):
        pltpu.async_copy(zerobuf, acc_sh.at[pl.ds(r0 + k * WB, WB)], sem_z)
    pltpu.sync_copy(zbuf1, deg_sh.at[pl.ds(r0, ROWS_PER_TILE)])

    cvec = jnp.broadcast_to(c, (16,)).astype(jnp.int32)

    def _xf(j, _):
        for v in range(CHUNK // 16):
            sl = pl.ds(v * 16, 16)
            rowb[j, sl] = rowb[j, sl] * 2 + cvec
        return _
    lax.fori_loop(0, TC_CHUNKS, _xf, None)

    for k in range(ROWS_PER_TILE // WB):
        pltpu.make_async_copy(zerobuf, acc_sh.at[pl.ds(r0, WB)],
                              sem_z).wait()
    plsc.subcore_barrier()

    ph1 = jax.named_scope("ph1_deg")
    ph1.__enter__()
    pltpu.sync_copy(col2d.at[pl.ds(cbase, SLAB)], colslab.at[0])

    def _deg(s, _):
        pb = (s % 2).astype(jnp.int32)

        @pl.when(jnp.logical_and(s > 0, pb == 0))
        def _():
            pltpu.make_async_copy(col2d.at[pl.ds(cbase, SLAB)],
                                  colslab.at[0], sem_c0).wait()

        @pl.when(jnp.logical_and(s > 0, pb == 1))
        def _():
            pltpu.make_async_copy(col2d.at[pl.ds(cbase, SLAB)],
                                  colslab.at[1], sem_c1).wait()

        @pl.when(jnp.logical_and(s < N_SLABS - 1, pb == 0))
        def _():
            pltpu.async_copy(col2d.at[pl.ds(cbase + (s + 1) * SLAB, SLAB)],
                             colslab.at[1], sem_c1)

        @pl.when(jnp.logical_and(s < N_SLABS - 1, pb == 1))
        def _():
            pltpu.async_copy(col2d.at[pl.ds(cbase + (s + 1) * SLAB, SLAB)],
                             colslab.at[0], sem_c0)
        for i in range(SLAB):
            pltpu.sync_copy(ones, deg_sh.at[colslab.at[pb, i]], add=True)
        return _
    lax.fori_loop(0, N_SLABS, _deg, None)
    plsc.subcore_barrier()
    ph1.__exit__(None, None, None)

    pltpu.sync_copy(deg_sh.at[pl.ds(r0, ROWS_PER_TILE)], zbuf1)

    def _rs(i, _):
        sl = pl.ds(i * 16, 16)
        zbuf1[sl] = _rsqrt16(zbuf1[sl])
        return _
    lax.fori_loop(0, ROWS_PER_TILE // 16, _rs, None)
    pltpu.sync_copy(zbuf1, deg_sh.at[pl.ds(r0, ROWS_PER_TILE)])
    plsc.subcore_barrier()
    pltpu.sync_copy(deg_sh, dinv_l)

    ph3 = jax.named_scope("ph3_main")
    ph3.__enter__()

    def _norm(j, pb, i):
        for v in range(CHUNK // 16):
            sl = pl.ds(v * 16, 16)
            ridx = lax.shift_right_arithmetic(rowb[j, sl] - cvec, 1)
            dr = plsc.load_gather(dinv_l, [ridx])
            dc = plsc.load_gather(dinv_l, [colslab[pb, i, sl]])
            normbuf[sl] = dr * ewb[j, sl] * dc

    def _scale(rows):
        @plsc.parallel_loop(0, CHUNK // 16, unroll=2)
        def _(g):
            nv16 = normbuf[pl.ds(g * 16, 16)]
            for l in range(16):
                nv = nv16[l]
                e = g * 16 + l
                for j in range(HD // 16):
                    sj = pl.ds(j * 16, 16)
                    rows[e, sj] = rows[e, sj] * nv

    pltpu.sync_copy(col2d.at[pl.ds(cbase, SLAB)], colslab.at[0])
    pltpu.async_copy(x2_hbm.at[rowb.at[0]], rows0, sem_g0)

    def _main(s, _):
        pb = (s % 2).astype(jnp.int32)

        @pl.when(jnp.logical_and(s > 0, pb == 0))
        def _():
            pltpu.make_async_copy(col2d.at[pl.ds(cbase, SLAB)],
                                  colslab.at[0], sem_c0).wait()

        @pl.when(jnp.logical_and(s > 0, pb == 1))
        def _():
            pltpu.make_async_copy(col2d.at[pl.ds(cbase, SLAB)],
                                  colslab.at[1], sem_c1).wait()
        for i in range(SLAB):
            ci = s * SLAB + i
            rows = rows0 if i % 2 == 0 else rows1
            other = rows1 if i % 2 == 0 else rows0
            sem_r = sem_g0 if i % 2 == 0 else sem_g1
            sem_o = sem_g1 if i % 2 == 0 else sem_g0
            sem_w = sem_s0 if i % 2 == 0 else sem_s1
            sem_v = sem_s1 if i % 2 == 0 else sem_s0
            _norm(ci, pb, i)
            pltpu.make_async_copy(x2_hbm.at[rowb.at[0]], rows, sem_r).wait()
            if i == 0:
                @pl.when(s > 0)
                def _():
                    pltpu.make_async_copy(other, acc_sh.at[colslab.at[0, 0]],
                                          sem_v).wait()
            else:
                pltpu.make_async_copy(other, acc_sh.at[colslab.at[0, 0]],
                                      sem_v).wait()
            if i < SLAB - 1:
                pltpu.async_copy(x2_hbm.at[rowb.at[ci + 1]], other, sem_o)
            else:
                @pl.when(s < N_SLABS - 1)
                def _():
                    pltpu.async_copy(x2_hbm.at[rowb.at[ci + 1]], other,
                                     sem_o)
            if i == 1:
                @pl.when(jnp.logical_and(s < N_SLABS - 1, pb == 0))
                def _():
                    pltpu.async_copy(
                        col2d.at[pl.ds(cbase + (s + 1) * SLAB, SLAB)],
                        colslab.at[1], sem_c1)

                @pl.when(jnp.logical_and(s < N_SLABS - 1, pb == 1))
                def _():
                    pltpu.async_copy(
                        col2d.at[pl.ds(cbase + (s + 1) * SLAB, SLAB)],
                        colslab.at[0], sem_c0)
            _scale(rows)
            pltpu.async_copy(rows, acc_sh.at[colslab.at[pb, i]], sem_w,
                             add=True)
        return _
    lax.fori_loop(0, N_SLABS, _main, None)
    pltpu.make_async_copy(rows1, acc_sh.at[colslab.at[0, 0]], sem_s1).wait()
    plsc.subcore_barrier()
    ph3.__exit__(None, None, None)

    hops = jnp.where(t < NS - 1, ROWS_PER_TILE // WB,
                     (N_NODES - (NS - 1) * ROWS_PER_TILE) // WB)

    def _wb(h, _):
        sl = pl.ds(r0 + h * WB, WB)
        pltpu.sync_copy(acc_sh.at[sl], zerobuf)
        pltpu.sync_copy(zerobuf, part_hbm.at[sl, c])
        return _
    lax.fori_loop(0, hops, _wb, None)


def _tc_mix_body(p_ref, o_ref):
    o_ref[:, :HD] = p_ref[0]
    o_ref[:, HD:] = p_ref[1]


_TC_R = 2000


@jax.jit
def kernel(x, edge_index, edge_weight):
    row = edge_index[0].astype(jnp.int32)
    col = edge_index[1].astype(jnp.int32)
    ew = edge_weight.astype(jnp.float32)

    npad = E_PAD - E
    ar = jnp.arange(npad, dtype=jnp.int32)
    row_p = jnp.concatenate([row, (ar * 97) % N_NODES])
    col_p = jnp.concatenate([col, N_NODES + ar % (N_PAD - N_NODES)])
    ew_p = jnp.concatenate([ew, jnp.zeros((npad,), jnp.float32)])
    row2d = row_p.reshape(N_CHUNKS, CHUNK)
    col2d = col_p.reshape(N_CHUNKS, CHUNK)
    ew2d = ew_p.reshape(N_CHUNKS, CHUNK)
    x2 = x.reshape(2 * N_NODES, HD)

    sc = pl.kernel(
        _sc_body,
        out_type=jax.ShapeDtypeStruct((N_NODES, NC, HD), jnp.float32),
        mesh=plsc.VectorSubcoreMesh(
            core_axis_name="c", subcore_axis_name="s",
            num_cores=NC, num_subcores=NS),
        compiler_params=pltpu.CompilerParams(
            needs_layout_passes=False, use_tc_tiling_on_sc=False),
        scratch_types=[
            pltpu.VMEM((TC_CHUNKS, CHUNK), jnp.int32),
            pltpu.VMEM((TC_CHUNKS, CHUNK), jnp.float32),
            pltpu.VMEM((2, SLAB, CHUNK), jnp.int32),
            pltpu.VMEM((CHUNK,), jnp.float32),
            pltpu.VMEM((CHUNK, HD), jnp.float32),
            pltpu.VMEM((CHUNK, HD), jnp.float32),
            pltpu.VMEM((N_PAD,), jnp.float32),
            pltpu.VMEM((CHUNK,), jnp.float32),
            pltpu.VMEM((WB, HD), jnp.float32),
            pltpu.VMEM((ROWS_PER_TILE,), jnp.float32),
            pltpu.VMEM_SHARED((N_PAD,), jnp.float32),
            pltpu.VMEM_SHARED((N_PAD, HD), jnp.float32),
            pltpu.SemaphoreType.DMA,
            pltpu.SemaphoreType.DMA,
            pltpu.SemaphoreType.DMA,
            pltpu.SemaphoreType.DMA,
            pltpu.SemaphoreType.DMA,
            pltpu.SemaphoreType.DMA,
            pltpu.SemaphoreType.DMA,
        ],
    )
    partials = sc(x2, row2d, col2d, ew2d)
    return partials.reshape(N_NODES, D)


# speedup vs baseline: 1.3967x; 1.3967x over previous
import jax
import jax.numpy as jnp
from jax import lax
from jax.experimental import pallas as pl
from jax.experimental.pallas import tpu as pltpu
from jax.experimental.pallas import tpu_sc as plsc

N_NODES = 10000
N_PAD = 10240
D = 128
HD = D // 2
E = 320000
CHUNK = 128
NC = 2
NS = 16
ROWS_PER_TILE = N_PAD // NS
WB = 80

E_PAD = 327680
N_CHUNKS = E_PAD // CHUNK
TC_CHUNKS = N_CHUNKS // NS
SLAB = 8
N_SLABS = TC_CHUNKS // SLAB
NBUF = 4


def _rsqrt16(d):
    ii = lax.bitcast_convert_type(d, jnp.int32)
    ii = jnp.int32(0x5F3759DF) - lax.shift_right_logical(ii, 1)
    y = lax.bitcast_convert_type(ii, jnp.float32)
    for _ in range(3):
        y = y * (1.5 - 0.5 * d * y * y)
    return jnp.where(d > 0.0, y, 0.0)


def _sc_body(x2_hbm, row2d, col2d, ew2d, part_hbm,
             rowb, ewslab, colslab, normbuf,
             rows0, rows1, rows2, rows3, dinv_l, ones, zerobuf, zbuf1,
             deg_sh, acc_sh,
             sem_g0, sem_g1, sem_g2, sem_g3,
             sem_s0, sem_s1, sem_s2, sem_s3,
             sem_c0, sem_c1, sem_e0, sem_e1, sem_z):
    c = lax.axis_index("c")
    t = lax.axis_index("s")
    r0 = t * ROWS_PER_TILE
    cbase = t * TC_CHUNKS
    rows_ring = (rows0, rows1, rows2, rows3)
    sem_g = (sem_g0, sem_g1, sem_g2, sem_g3)
    sem_s = (sem_s0, sem_s1, sem_s2, sem_s3)

    zvec = jnp.zeros((16,), jnp.float32)
    ovec = jnp.ones((16,), jnp.float32)
    for j in range(CHUNK // 16):
        ones[pl.ds(j * 16, 16)] = ovec

    def _zrow(i, _):
        for j in range(HD // 16):
            zerobuf[i, pl.ds(j * 16, 16)] = zvec
        return _
    lax.fori_loop(0, WB, _zrow, None)

    def _z1(i, _):
        zbuf1[pl.ds(i * 16, 16)] = zvec
        return _
    lax.fori_loop(0, ROWS_PER_TILE // 16, _z1, None)

    pltpu.sync_copy(row2d.at[pl.ds(cbase, TC_CHUNKS)], rowb)

    for k in range(ROWS_PER_TILE // WB):
        pltpu.async_copy(zerobuf, acc_sh.at[pl.ds(r0 + k * WB, WB)], sem_z)
    pltpu.sync_copy(zbuf1, deg_sh.at[pl.ds(r0, ROWS_PER_TILE)])

    cvec = jnp.broadcast_to(c, (16,)).astype(jnp.int32)

    def _xf(j, _):
        for v in range(CHUNK // 16):
            sl = pl.ds(v * 16, 16)
            rowb[j, sl] = rowb[j, sl] * 2 + cvec
        return _
    lax.fori_loop(0, TC_CHUNKS, _xf, None)

    for k in range(ROWS_PER_TILE // WB):
        pltpu.make_async_copy(zerobuf, acc_sh.at[pl.ds(r0, WB)],
                              sem_z).wait()
    plsc.subcore_barrier()

    ph1 = jax.named_scope("ph1_deg")
    ph1.__enter__()
    pltpu.sync_copy(col2d.at[pl.ds(cbase, SLAB)], colslab.at[0])

    def _deg(s, _):
        pb = (s % 2).astype(jnp.int32)

        @pl.when(jnp.logical_and(s > 0, pb == 0))
        def _():
            pltpu.make_async_copy(col2d.at[pl.ds(cbase, SLAB)],
                                  colslab.at[0], sem_c0).wait()

        @pl.when(jnp.logical_and(s > 0, pb == 1))
        def _():
            pltpu.make_async_copy(col2d.at[pl.ds(cbase, SLAB)],
                                  colslab.at[1], sem_c1).wait()

        @pl.when(jnp.logical_and(s < N_SLABS - 1, pb == 0))
        def _():
            pltpu.async_copy(col2d.at[pl.ds(cbase + (s + 1) * SLAB, SLAB)],
                             colslab.at[1], sem_c1)

        @pl.when(jnp.logical_and(s < N_SLABS - 1, pb == 1))
        def _():
            pltpu.async_copy(col2d.at[pl.ds(cbase + (s + 1) * SLAB, SLAB)],
                             colslab.at[0], sem_c0)
        for i in range(SLAB):
            pltpu.sync_copy(ones, deg_sh.at[colslab.at[pb, i]], add=True)
        return _
    lax.fori_loop(0, N_SLABS, _deg, None)
    plsc.subcore_barrier()
    ph1.__exit__(None, None, None)

    pltpu.sync_copy(deg_sh.at[pl.ds(r0, ROWS_PER_TILE)], zbuf1)

    def _rs(i, _):
        sl = pl.ds(i * 16, 16)
        zbuf1[sl] = _rsqrt16(zbuf1[sl])
        return _
    lax.fori_loop(0, ROWS_PER_TILE // 16, _rs, None)
    pltpu.sync_copy(zbuf1, deg_sh.at[pl.ds(r0, ROWS_PER_TILE)])
    plsc.subcore_barrier()
    pltpu.sync_copy(deg_sh, dinv_l)

    ph3 = jax.named_scope("ph3_main")
    ph3.__enter__()

    def _norm(j, pb, i):
        for v in range(CHUNK // 16):
            sl = pl.ds(v * 16, 16)
            ridx = lax.shift_right_arithmetic(rowb[j, sl] - cvec, 1)
            dr = plsc.load_gather(dinv_l, [ridx])
            dc = plsc.load_gather(dinv_l, [colslab[pb, i, sl]])
            normbuf[sl] = dr * ewslab[pb, i, sl] * dc

    def _scale(rows):
        @plsc.parallel_loop(0, CHUNK // 16, unroll=2)
        def _(g):
            nv16 = normbuf[pl.ds(g * 16, 16)]
            for l in range(16):
                nv = nv16[l]
                e = g * 16 + l
                for j in range(HD // 16):
                    sj = pl.ds(j * 16, 16)
                    rows[e, sj] = rows[e, sj] * nv

    pltpu.sync_copy(col2d.at[pl.ds(cbase, SLAB)], colslab.at[0])
    pltpu.sync_copy(ew2d.at[pl.ds(cbase, SLAB)], ewslab.at[0])
    for b in range(NBUF - 1):
        pltpu.async_copy(x2_hbm.at[rowb.at[b]], rows_ring[b], sem_g[b])

    def _main(s, _):
        pb = (s % 2).astype(jnp.int32)

        @pl.when(jnp.logical_and(s > 0, pb == 0))
        def _():
            pltpu.make_async_copy(col2d.at[pl.ds(cbase, SLAB)],
                                  colslab.at[0], sem_c0).wait()
            pltpu.make_async_copy(ew2d.at[pl.ds(cbase, SLAB)],
                                  ewslab.at[0], sem_e0).wait()

        @pl.when(jnp.logical_and(s > 0, pb == 1))
        def _():
            pltpu.make_async_copy(col2d.at[pl.ds(cbase, SLAB)],
                                  colslab.at[1], sem_c1).wait()
            pltpu.make_async_copy(ew2d.at[pl.ds(cbase, SLAB)],
                                  ewslab.at[1], sem_e1).wait()
        for i in range(SLAB):
            ci = s * SLAB + i
            b = i % NBUF
            b3 = (i + 3) % NBUF
            rows = rows_ring[b]
            _norm(ci, pb, i)
            pltpu.make_async_copy(x2_hbm.at[rowb.at[0]], rows,
                                  sem_g[b]).wait()
            _scale(rows)
            pltpu.async_copy(rows, acc_sh.at[colslab.at[pb, i]], sem_s[b],
                             add=True)
            if i == 0:
                @pl.when(s > 0)
                def _():
                    pltpu.make_async_copy(rows_ring[b3],
                                          acc_sh.at[colslab.at[0, 0]],
                                          sem_s[b3]).wait()
            else:
                pltpu.make_async_copy(rows_ring[b3],
                                      acc_sh.at[colslab.at[0, 0]],
                                      sem_s[b3]).wait()
            if i <= 4:
                pltpu.async_copy(x2_hbm.at[rowb.at[ci + 3]], rows_ring[b3],
                                 sem_g[b3])
            else:
                @pl.when(s < N_SLABS - 1)
                def _():
                    pltpu.async_copy(x2_hbm.at[rowb.at[ci + 3]],
                                     rows_ring[b3], sem_g[b3])
            if i == 1:
                @pl.when(jnp.logical_and(s < N_SLABS - 1, pb == 0))
                def _():
                    pltpu.async_copy(
                        col2d.at[pl.ds(cbase + (s + 1) * SLAB, SLAB)],
                        colslab.at[1], sem_c1)
                    pltpu.async_copy(
                        ew2d.at[pl.ds(cbase + (s + 1) * SLAB, SLAB)],
                        ewslab.at[1], sem_e1)

                @pl.when(jnp.logical_and(s < N_SLABS - 1, pb == 1))
                def _():
                    pltpu.async_copy(
                        col2d.at[pl.ds(cbase + (s + 1) * SLAB, SLAB)],
                        colslab.at[0], sem_c0)
                    pltpu.async_copy(
                        ew2d.at[pl.ds(cbase + (s + 1) * SLAB, SLAB)],
                        ewslab.at[0], sem_e0)
        return _
    lax.fori_loop(0, N_SLABS, _main, None)
    pltpu.make_async_copy(rows3, acc_sh.at[colslab.at[0, 0]], sem_s3).wait()
    plsc.subcore_barrier()
    ph3.__exit__(None, None, None)

    hops = jnp.where(t < NS - 1, ROWS_PER_TILE // WB,
                     (N_NODES - (NS - 1) * ROWS_PER_TILE) // WB)

    def _wb(h, _):
        sl = pl.ds(r0 + h * WB, WB)
        pltpu.sync_copy(acc_sh.at[sl], zerobuf)
        pltpu.sync_copy(zerobuf, part_hbm.at[c, sl])
        return _
    lax.fori_loop(0, hops, _wb, None)


def _tc_mix_body(p_ref, o_ref):
    o_ref[:, :HD] = p_ref[0]
    o_ref[:, HD:] = p_ref[1]


_TC_R = 2000


@jax.jit
def kernel(x, edge_index, edge_weight):
    row = edge_index[0].astype(jnp.int32)
    col = edge_index[1].astype(jnp.int32)
    ew = edge_weight.astype(jnp.float32)

    npad = E_PAD - E
    ar = jnp.arange(npad, dtype=jnp.int32)
    row_p = jnp.concatenate([row, (ar * 97) % N_NODES])
    col_p = jnp.concatenate([col, N_NODES + ar % (N_PAD - N_NODES)])
    ew_p = jnp.concatenate([ew, jnp.zeros((npad,), jnp.float32)])
    row2d = row_p.reshape(N_CHUNKS, CHUNK)
    col2d = col_p.reshape(N_CHUNKS, CHUNK)
    ew2d = ew_p.reshape(N_CHUNKS, CHUNK)
    x2 = x.reshape(2 * N_NODES, HD)

    sc = pl.kernel(
        _sc_body,
        out_type=jax.ShapeDtypeStruct((NC, N_NODES, HD), jnp.float32),
        mesh=plsc.VectorSubcoreMesh(
            core_axis_name="c", subcore_axis_name="s",
            num_cores=NC, num_subcores=NS),
        compiler_params=pltpu.CompilerParams(
            needs_layout_passes=False, use_tc_tiling_on_sc=False),
        scratch_types=[
            pltpu.VMEM((TC_CHUNKS, CHUNK), jnp.int32),
            pltpu.VMEM((2, SLAB, CHUNK), jnp.float32),
            pltpu.VMEM((2, SLAB, CHUNK), jnp.int32),
            pltpu.VMEM((CHUNK,), jnp.float32),
            pltpu.VMEM((CHUNK, HD), jnp.float32),
            pltpu.VMEM((CHUNK, HD), jnp.float32),
            pltpu.VMEM((CHUNK, HD), jnp.float32),
            pltpu.VMEM((CHUNK, HD), jnp.float32),
            pltpu.VMEM((N_PAD,), jnp.float32),
            pltpu.VMEM((CHUNK,), jnp.float32),
            pltpu.VMEM((WB, HD), jnp.float32),
            pltpu.VMEM((ROWS_PER_TILE,), jnp.float32),
            pltpu.VMEM_SHARED((N_PAD,), jnp.float32),
            pltpu.VMEM_SHARED((N_PAD, HD), jnp.float32),
        ] + [pltpu.SemaphoreType.DMA] * 13,
    )
    partials = sc(x2, row2d, col2d, ew2d)

    out = pl.pallas_call(
        _tc_mix_body,
        grid=(N_NODES // _TC_R,),
        in_specs=[pl.BlockSpec((NC, _TC_R, HD), lambda i: (0, i, 0))],
        out_specs=pl.BlockSpec((_TC_R, D), lambda i: (i, 0)),
        out_shape=jax.ShapeDtypeStruct((N_NODES, D), jnp.float32),
    )(partials)
    return out
